# Initial kernel scaffold; baseline (speedup 1.0000x reference)
import jax
import jax.numpy as jnp
from jax.experimental import pallas as pl


def kernel(z, emb_weight):
    raise NotImplementedError("write your pallas kernel here")


# speedup vs baseline: 6.4201x; 6.4201x over previous
import jax
import jax.numpy as jnp
from jax import lax
from jax.experimental import pallas as pl
from jax.experimental.pallas import tpu as pltpu

NUM_EMB = 8192
DIM = 256
ROW_BLK = 2048
COL_BLK = 1024
NCOL = NUM_EMB // COL_BLK
SEG_BOUNDS = (2736, 5472, 8192)


def _bnd_for_block(j):
    return jnp.where(j < 3, SEG_BOUNDS[0], jnp.where(j < 6, SEG_BOUNDS[1], SEG_BOUNDS[2]))


def _minarg(d, gcol):
    lmin = jnp.min(d, axis=1, keepdims=True)
    larg = jnp.min(jnp.where(d == lmin, gcol, NUM_EMB), axis=1, keepdims=True)
    return lmin, larg


def _dist_body(z_ref, e_ref, z2_ref, e2_ref,
               lminL_ref, largL_ref, lminH_ref, largH_ref):
    j = pl.program_id(1)
    mm = lax.dot_general(
        z_ref[...], e_ref[...],
        (((1,), (1,)), ((), ())),
        preferred_element_type=jnp.float32,
    )
    d = (z2_ref[...] - 2.0 * mm) + e2_ref[...]
    gcol = j * COL_BLK + lax.broadcasted_iota(jnp.int32, (1, COL_BLK), 1)
    bnd = _bnd_for_block(j)
    mask = gcol < bnd
    inf = jnp.float32(jnp.inf)
    d_lo = jnp.where(mask, d, inf)
    lmin, larg = _minarg(d_lo, gcol)
    lminL_ref[0] = lmin
    largL_ref[0] = larg
    straddle = jnp.logical_or(j == 2, j == 5)

    @pl.when(straddle)
    def _():
        d_hi = jnp.where(mask, inf, d)
        lmin_h, larg_h = _minarg(d_hi, gcol)
        lminH_ref[0] = lmin_h
        largH_ref[0] = larg_h

    @pl.when(jnp.logical_not(straddle))
    def _():
        lminH_ref[0] = jnp.full((ROW_BLK, 1), inf, jnp.float32)
        largH_ref[0] = jnp.zeros((ROW_BLK, 1), jnp.int32)


def _dist_call(z_flat, emb_weight, z2, e2):
    n = z_flat.shape[0]
    grid = (n // ROW_BLK, NCOL)
    out = pl.pallas_call(
        _dist_body,
        grid=grid,
        in_specs=[
            pl.BlockSpec((ROW_BLK, DIM), lambda i, j: (i, 0)),
            pl.BlockSpec((COL_BLK, DIM), lambda i, j: (j, 0)),
            pl.BlockSpec((ROW_BLK, 1), lambda i, j: (i, 0)),
            pl.BlockSpec((1, COL_BLK), lambda i, j: (0, j)),
        ],
        out_specs=[
            pl.BlockSpec((1, ROW_BLK, 1), lambda i, j: (j, i, 0)),
            pl.BlockSpec((1, ROW_BLK, 1), lambda i, j: (j, i, 0)),
            pl.BlockSpec((1, ROW_BLK, 1), lambda i, j: (j, i, 0)),
            pl.BlockSpec((1, ROW_BLK, 1), lambda i, j: (j, i, 0)),
        ],
        out_shape=[
            jax.ShapeDtypeStruct((NCOL, n, 1), jnp.float32),
            jax.ShapeDtypeStruct((NCOL, n, 1), jnp.int32),
            jax.ShapeDtypeStruct((NCOL, n, 1), jnp.float32),
            jax.ShapeDtypeStruct((NCOL, n, 1), jnp.int32),
        ],
    )(z_flat, emb_weight, z2, e2)
    return out


def _comb(av, ai, ae, ev, ei, ee):
    keep = jnp.logical_or(av < ev, jnp.logical_and(av == ev, ai < ei))
    return (jnp.where(keep, av, ev), jnp.where(keep, ai, ei),
            jnp.where(keep, ae, ee))


def _bf16_round(v):
    return lax.reduce_precision(v, exponent_bits=8, mantissa_bits=7)


def kernel(z, emb_weight):
    b, c, h, w = z.shape
    z_perm = jnp.transpose(z, (0, 2, 3, 1))
    z_flat = z_perm.reshape(-1, DIM)
    z2 = jnp.sum(z_perm ** 2, axis=-1).reshape(-1, 1)
    e2 = jnp.sum(emb_weight ** 2, axis=1).reshape(1, NUM_EMB)
    lminL, largL, lminH, largH = _dist_call(z_flat, emb_weight, z2, e2)

    L = lambda k: (lminL[k, :, 0], largL[k, :, 0])
    H = lambda k: (lminH[k, :, 0], largH[k, :, 0])
    seg_parts = (
        (L(0), L(1), L(2)),
        (H(2), L(3), L(4), L(5)),
        (H(5), L(6), L(7)),
    )
    segs = []
    for parts in seg_parts:
        sv, si = parts[0]
        for ev, ei in parts[1:]:
            keep = jnp.logical_or(sv < ev, jnp.logical_and(sv == ev, si < ei))
            sv = jnp.where(keep, sv, ev)
            si = jnp.where(keep, si, ei)
        segs.append((sv, si))

    av, ai, ae = _bf16_round(segs[0][0]), segs[0][1], segs[0][0]
    for sv, si in segs[1:]:
        av, ai, ae = _comb(av, ai, ae, sv, si, sv)
        av = _bf16_round(av)

    idx = ai.astype(jnp.int32)[:, None]
    quantized = jnp.take(emb_weight, ai, axis=0)
    loss = 1.25 * (jnp.sum(ae) / jnp.float32(z.size))
    qout = jnp.transpose(quantized.reshape(b, h, w, c), (0, 3, 1, 2))
    return qout, loss, idx


# speedup vs baseline: 7.2052x; 1.1223x over previous
import functools

import jax
import jax.numpy as jnp
from jax import lax
from jax.experimental import pallas as pl
from jax.experimental.pallas import tpu as pltpu
from jax.experimental.pallas import tpu_sc as plsc

NUM_EMB = 8192
DIM = 256
ROW_BLK = 2048
COL_BLK = 1024
NCOL = NUM_EMB // COL_BLK
SEG_BOUNDS = (2736, 5472, 8192)


def _bnd_for_block(j):
    return jnp.where(j < 3, SEG_BOUNDS[0], jnp.where(j < 6, SEG_BOUNDS[1], SEG_BOUNDS[2]))


def _minarg(d, gcol):
    lmin = jnp.min(d, axis=1, keepdims=True)
    larg = jnp.min(jnp.where(d == lmin, gcol, NUM_EMB), axis=1, keepdims=True)
    return lmin, larg


def _dist_body(z_ref, e_ref, z2_ref, e2_ref,
               lminL_ref, largL_ref, lminH_ref, largH_ref):
    j = pl.program_id(1)
    mm = lax.dot_general(
        z_ref[...], e_ref[...],
        (((1,), (1,)), ((), ())),
        preferred_element_type=jnp.float32,
    )
    d = (z2_ref[...] - 2.0 * mm) + e2_ref[...]
    gcol = j * COL_BLK + lax.broadcasted_iota(jnp.int32, (1, COL_BLK), 1)
    bnd = _bnd_for_block(j)
    mask = gcol < bnd
    inf = jnp.float32(jnp.inf)
    d_lo = jnp.where(mask, d, inf)
    lmin, larg = _minarg(d_lo, gcol)
    lminL_ref[0] = lmin
    largL_ref[0] = larg
    straddle = jnp.logical_or(j == 2, j == 5)

    @pl.when(straddle)
    def _():
        d_hi = jnp.where(mask, inf, d)
        lmin_h, larg_h = _minarg(d_hi, gcol)
        lminH_ref[0] = lmin_h
        largH_ref[0] = larg_h

    @pl.when(jnp.logical_not(straddle))
    def _():
        lminH_ref[0] = jnp.full((ROW_BLK, 1), inf, jnp.float32)
        largH_ref[0] = jnp.zeros((ROW_BLK, 1), jnp.int32)


def _dist_call(z_flat, emb_weight, z2, e2):
    n = z_flat.shape[0]
    grid = (n // ROW_BLK, NCOL)
    out = pl.pallas_call(
        _dist_body,
        grid=grid,
        in_specs=[
            pl.BlockSpec((ROW_BLK, DIM), lambda i, j: (i, 0)),
            pl.BlockSpec((COL_BLK, DIM), lambda i, j: (j, 0)),
            pl.BlockSpec((ROW_BLK, 1), lambda i, j: (i, 0)),
            pl.BlockSpec((1, COL_BLK), lambda i, j: (0, j)),
        ],
        out_specs=[
            pl.BlockSpec((1, ROW_BLK, 1), lambda i, j: (j, i, 0)),
            pl.BlockSpec((1, ROW_BLK, 1), lambda i, j: (j, i, 0)),
            pl.BlockSpec((1, ROW_BLK, 1), lambda i, j: (j, i, 0)),
            pl.BlockSpec((1, ROW_BLK, 1), lambda i, j: (j, i, 0)),
        ],
        out_shape=[
            jax.ShapeDtypeStruct((NCOL, n, 1), jnp.float32),
            jax.ShapeDtypeStruct((NCOL, n, 1), jnp.int32),
            jax.ShapeDtypeStruct((NCOL, n, 1), jnp.float32),
            jax.ShapeDtypeStruct((NCOL, n, 1), jnp.int32),
        ],
    )(z_flat, emb_weight, z2, e2)
    return out


def _comb(av, ai, ae, ev, ei, ee):
    keep = jnp.logical_or(av < ev, jnp.logical_and(av == ev, ai < ei))
    return (jnp.where(keep, av, ev), jnp.where(keep, ai, ei),
            jnp.where(keep, ae, ee))


_GATHER_N = 16384
_GATHER_NW = 32
_GATHER_NC = 2
_GATHER_CH = 128


def _gather_call(emb_weight, idx):
    b_per_w = _GATHER_N // _GATHER_NW
    mesh = plsc.VectorSubcoreMesh(core_axis_name="c", subcore_axis_name="s")

    @functools.partial(
        pl.kernel, mesh=mesh,
        out_type=jax.ShapeDtypeStruct((_GATHER_N, DIM), jnp.float32),
        scratch_types=[
            pltpu.VMEM((_GATHER_CH,), jnp.int32),
            pltpu.VMEM((_GATHER_CH, DIM), jnp.float32),
            pltpu.SemaphoreType.DMA,
        ],
    )
    def gk(table_hbm, idx_hbm, out_hbm, idx_v, rows_v, sem):
        wid = lax.axis_index("s") * _GATHER_NC + lax.axis_index("c")
        base = wid * b_per_w

        def body(c, carry):
            off = base + c * _GATHER_CH
            pltpu.sync_copy(idx_hbm.at[pl.ds(off, _GATHER_CH)], idx_v)
            pltpu.async_copy(table_hbm.at[idx_v], rows_v, sem).wait()
            pltpu.sync_copy(rows_v, out_hbm.at[pl.ds(off, _GATHER_CH)])
            return carry

        lax.fori_loop(0, b_per_w // _GATHER_CH, body, 0)

    return gk(emb_weight, idx)


def _bf16_round(v):
    return lax.reduce_precision(v, exponent_bits=8, mantissa_bits=7)


def kernel(z, emb_weight):
    b, c, h, w = z.shape
    z_perm = jnp.transpose(z, (0, 2, 3, 1))
    z_flat = z_perm.reshape(-1, DIM)
    z2 = jnp.sum(z_perm ** 2, axis=-1).reshape(-1, 1)
    e2 = jnp.sum(emb_weight ** 2, axis=1).reshape(1, NUM_EMB)
    lminL, largL, lminH, largH = _dist_call(z_flat, emb_weight, z2, e2)

    L = lambda k: (lminL[k, :, 0], largL[k, :, 0])
    H = lambda k: (lminH[k, :, 0], largH[k, :, 0])
    seg_parts = (
        (L(0), L(1), L(2)),
        (H(2), L(3), L(4), L(5)),
        (H(5), L(6), L(7)),
    )
    segs = []
    for parts in seg_parts:
        sv, si = parts[0]
        for ev, ei in parts[1:]:
            keep = jnp.logical_or(sv < ev, jnp.logical_and(sv == ev, si < ei))
            sv = jnp.where(keep, sv, ev)
            si = jnp.where(keep, si, ei)
        segs.append((sv, si))

    av, ai, ae = _bf16_round(segs[0][0]), segs[0][1], segs[0][0]
    for sv, si in segs[1:]:
        av, ai, ae = _comb(av, ai, ae, sv, si, sv)
        av = _bf16_round(av)

    idx_flat = ai.astype(jnp.int32)
    idx = idx_flat[:, None]
    quantized = _gather_call(emb_weight, idx_flat)
    loss = 1.25 * (jnp.sum(ae) / jnp.float32(z.size))
    qout = jnp.transpose(quantized.reshape(b, h, w, c), (0, 3, 1, 2))
    return qout, loss, idx


# speedup vs baseline: 7.5850x; 1.0527x over previous
import functools

import jax
import jax.numpy as jnp
from jax import lax
from jax.experimental import pallas as pl
from jax.experimental.pallas import tpu as pltpu
from jax.experimental.pallas import tpu_sc as plsc

NUM_EMB = 8192
DIM = 256
ROW_BLK = 2048
COL_BLK = 1024
NCOL = NUM_EMB // COL_BLK
SEG_BOUNDS = (2736, 5472, 8192)


def _bnd_for_block(j):
    return jnp.where(j < 3, SEG_BOUNDS[0], jnp.where(j < 6, SEG_BOUNDS[1], SEG_BOUNDS[2]))


def _minarg(d, gcol):
    lmin = jnp.min(d, axis=1, keepdims=True)
    larg = jnp.min(jnp.where(d == lmin, gcol, NUM_EMB), axis=1, keepdims=True)
    return lmin, larg


def _dist_body(z_ref, e_ref, z2_ref, e2_ref,
               lminL_ref, largL_ref, lminH_ref, largH_ref):
    j = pl.program_id(1)
    mm = lax.dot_general(
        z_ref[...], e_ref[...],
        (((1,), (1,)), ((), ())),
        preferred_element_type=jnp.float32,
    )
    d = (z2_ref[...] - 2.0 * mm) + e2_ref[...]
    gcol = j * COL_BLK + lax.broadcasted_iota(jnp.int32, (1, COL_BLK), 1)
    inf = jnp.float32(jnp.inf)
    straddle = jnp.logical_or(j == 2, j == 5)

    @pl.when(straddle)
    def _():
        bnd = _bnd_for_block(j)
        mask = gcol < bnd
        lmin, larg = _minarg(jnp.where(mask, d, inf), gcol)
        lminL_ref[0] = lmin
        largL_ref[0] = larg
        lmin_h, larg_h = _minarg(jnp.where(mask, inf, d), gcol)
        lminH_ref[0] = lmin_h
        largH_ref[0] = larg_h

    @pl.when(jnp.logical_not(straddle))
    def _():
        lmin, larg = _minarg(d, gcol)
        lminL_ref[0] = lmin
        largL_ref[0] = larg
        lminH_ref[0] = jnp.full((ROW_BLK, 1), inf, jnp.float32)
        largH_ref[0] = jnp.zeros((ROW_BLK, 1), jnp.int32)


def _dist_call(z_flat, emb_weight, z2, e2):
    n = z_flat.shape[0]
    grid = (n // ROW_BLK, NCOL)
    out = pl.pallas_call(
        _dist_body,
        grid=grid,
        in_specs=[
            pl.BlockSpec((ROW_BLK, DIM), lambda i, j: (i, 0)),
            pl.BlockSpec((COL_BLK, DIM), lambda i, j: (j, 0)),
            pl.BlockSpec((ROW_BLK, 1), lambda i, j: (i, 0)),
            pl.BlockSpec((1, COL_BLK), lambda i, j: (0, j)),
        ],
        out_specs=[
            pl.BlockSpec((1, ROW_BLK, 1), lambda i, j: (j, i, 0)),
            pl.BlockSpec((1, ROW_BLK, 1), lambda i, j: (j, i, 0)),
            pl.BlockSpec((1, ROW_BLK, 1), lambda i, j: (j, i, 0)),
            pl.BlockSpec((1, ROW_BLK, 1), lambda i, j: (j, i, 0)),
        ],
        out_shape=[
            jax.ShapeDtypeStruct((NCOL, n, 1), jnp.float32),
            jax.ShapeDtypeStruct((NCOL, n, 1), jnp.int32),
            jax.ShapeDtypeStruct((NCOL, n, 1), jnp.float32),
            jax.ShapeDtypeStruct((NCOL, n, 1), jnp.int32),
        ],
    )(z_flat, emb_weight, z2, e2)
    return out


def _comb(av, ai, ae, ev, ei, ee):
    keep = jnp.logical_or(av < ev, jnp.logical_and(av == ev, ai < ei))
    return (jnp.where(keep, av, ev), jnp.where(keep, ai, ei),
            jnp.where(keep, ae, ee))


_GATHER_N = 16384
_GATHER_NW = 32
_GATHER_NC = 2
_GATHER_CH = 128


def _gather_call(emb_weight, idx):
    b_per_w = _GATHER_N // _GATHER_NW
    mesh = plsc.VectorSubcoreMesh(core_axis_name="c", subcore_axis_name="s")

    @functools.partial(
        pl.kernel, mesh=mesh,
        out_type=jax.ShapeDtypeStruct((_GATHER_N, DIM), jnp.float32),
        scratch_types=[
            pltpu.VMEM((_GATHER_CH,), jnp.int32),
            pltpu.VMEM((_GATHER_CH, DIM), jnp.float32),
            pltpu.SemaphoreType.DMA,
        ],
    )
    def gk(table_hbm, idx_hbm, out_hbm, idx_v, rows_v, sem):
        wid = lax.axis_index("s") * _GATHER_NC + lax.axis_index("c")
        base = wid * b_per_w

        def body(c, carry):
            off = base + c * _GATHER_CH
            pltpu.sync_copy(idx_hbm.at[pl.ds(off, _GATHER_CH)], idx_v)
            pltpu.async_copy(table_hbm.at[idx_v], rows_v, sem).wait()
            pltpu.sync_copy(rows_v, out_hbm.at[pl.ds(off, _GATHER_CH)])
            return carry

        lax.fori_loop(0, b_per_w // _GATHER_CH, body, 0)

    return gk(emb_weight, idx)


def _bf16_round(v):
    return lax.reduce_precision(v, exponent_bits=8, mantissa_bits=7)


def kernel(z, emb_weight):
    b, c, h, w = z.shape
    z_perm = jnp.transpose(z, (0, 2, 3, 1))
    z_flat = z_perm.reshape(-1, DIM)
    z2 = jnp.sum(z_perm ** 2, axis=-1).reshape(-1, 1)
    e2 = jnp.sum(emb_weight ** 2, axis=1).reshape(1, NUM_EMB)
    lminL, largL, lminH, largH = _dist_call(z_flat, emb_weight, z2, e2)

    L = lambda k: (lminL[k, :, 0], largL[k, :, 0])
    H = lambda k: (lminH[k, :, 0], largH[k, :, 0])
    seg_parts = (
        (L(0), L(1), L(2)),
        (H(2), L(3), L(4), L(5)),
        (H(5), L(6), L(7)),
    )
    segs = []
    for parts in seg_parts:
        sv, si = parts[0]
        for ev, ei in parts[1:]:
            keep = jnp.logical_or(sv < ev, jnp.logical_and(sv == ev, si < ei))
            sv = jnp.where(keep, sv, ev)
            si = jnp.where(keep, si, ei)
        segs.append((sv, si))

    av, ai, ae = _bf16_round(segs[0][0]), segs[0][1], segs[0][0]
    for sv, si in segs[1:]:
        av, ai, ae = _comb(av, ai, ae, sv, si, sv)
        av = _bf16_round(av)

    idx_flat = ai.astype(jnp.int32)
    idx = idx_flat[:, None]
    quantized = _gather_call(emb_weight, idx_flat)
    loss = 1.25 * (jnp.sum(ae) / jnp.float32(z.size))
    qout = jnp.transpose(quantized.reshape(b, h, w, c), (0, 3, 1, 2))
    return qout, loss, idx
